# baseline (speedup 1.0000x reference)
import functools

import jax
import jax.numpy as jnp
from jax import lax
from jax.experimental import pallas as pl
from jax.experimental.pallas import tpu as pltpu
from jax.experimental.pallas import tpu_sc as plsc

K = 20
TL = 1024


def _k1s_body(x_ref, wlT_ref, bl_ref, wl1T_ref, bl1_ref,
              wmT_ref, bm_ref, waT_ref, ba_ref,
              g1w_ref, g1b_ref, g2w_ref, g2b_ref, agg_ref, aggT_ref,
              s_ref, x2_s, sum_s, max_s, qT_s, *, G, NT, L, groups, eps):
    i = pl.program_id(0)
    t = i % NT
    cs = 1.0 / (agg_ref.shape[0] // groups)
    hi = lax.Precision.HIGHEST

    @pl.when((t == 0) & (i >= NT))
    def _():
        def gn(y, w, b_):
            gs = jnp.dot(y, agg_ref[...], precision=hi,
                         preferred_element_type=jnp.float32) * cs
            gss = jnp.dot(y * y, agg_ref[...], precision=hi,
                          preferred_element_type=jnp.float32) * cs
            rstd = lax.rsqrt(gss - gs * gs + eps)
            mean_f = jnp.dot(gs, aggT_ref[...], precision=hi,
                             preferred_element_type=jnp.float32)
            rstd_f = jnp.dot(rstd, aggT_ref[...], precision=hi,
                             preferred_element_type=jnp.float32)
            return (y - mean_f) * rstd_f * w + b_

        mean = (sum_s[...] * (1.0 / L)).astype(jnp.bfloat16)
        xm = jnp.dot(mean, wmT_ref[...],
                     preferred_element_type=jnp.float32) + bm_ref[...]
        xa = jnp.dot(max_s[...].astype(jnp.bfloat16), waT_ref[...],
                     preferred_element_type=jnp.float32) + ba_ref[...]
        D = xm.shape[1]
        g1 = gn(xm, g1w_ref[...], g1b_ref[...]).astype(jnp.bfloat16)
        g2 = gn(xa, g2w_ref[...], g2b_ref[...]).astype(jnp.bfloat16)
        qT_s[...] = jnp.concatenate(
            [jnp.swapaxes(g1, 0, 1), jnp.swapaxes(g2, 0, 1),
             jnp.zeros((D, 126), jnp.bfloat16)], axis=1)

    @pl.when(i >= NT)
    def _():
        s128 = jnp.dot(x2_s[t], qT_s[...], preferred_element_type=jnp.float32)
        s_ref[0] = s128[:, 0:1] + s128[:, 1:2]

    @pl.when(i < G)
    def _():
        x1 = jnp.dot(x_ref[0].astype(jnp.bfloat16), wlT_ref[...],
                     preferred_element_type=jnp.float32) + bl_ref[...]
        part_sum = jnp.sum(x1, axis=0, keepdims=True)
        part_max = jnp.max(x1, axis=0, keepdims=True)

        @pl.when(t == 0)
        def _():
            sum_s[...] = part_sum
            max_s[...] = part_max

        @pl.when(t != 0)
        def _():
            sum_s[...] = sum_s[...] + part_sum
            max_s[...] = jnp.maximum(max_s[...], part_max)

        x2 = jnp.dot(x1.astype(jnp.bfloat16), wl1T_ref[...],
                     preferred_element_type=jnp.float32) + bl1_ref[...]
        x2_s[t] = x2.astype(jnp.bfloat16)


def _k34_body(x2_ref, sum_ref, max_ref, wmT_ref, bm_ref, waT_ref, ba_ref,
              g1w_ref, g1b_ref, g2w_ref, g2b_ref, agg_ref, aggT_ref,
              s_ref, qT_s, *, L, groups, eps):
    t = pl.program_id(1)
    cs = 1.0 / (agg_ref.shape[0] // groups)
    hi = lax.Precision.HIGHEST

    @pl.when(t == 0)
    def _():
        def gn(y, w, b_):
            gs = jnp.dot(y, agg_ref[...], precision=hi,
                         preferred_element_type=jnp.float32) * cs
            gss = jnp.dot(y * y, agg_ref[...], precision=hi,
                          preferred_element_type=jnp.float32) * cs
            rstd = lax.rsqrt(gss - gs * gs + eps)
            mean_f = jnp.dot(gs, aggT_ref[...], precision=hi,
                             preferred_element_type=jnp.float32)
            rstd_f = jnp.dot(rstd, aggT_ref[...], precision=hi,
                             preferred_element_type=jnp.float32)
            return (y - mean_f) * rstd_f * w + b_

        mean = (sum_ref[0] * (1.0 / L)).astype(jnp.bfloat16)
        xm = jnp.dot(mean, wmT_ref[...],
                     preferred_element_type=jnp.float32) + bm_ref[...]
        xa = jnp.dot(max_ref[0].astype(jnp.bfloat16), waT_ref[...],
                     preferred_element_type=jnp.float32) + ba_ref[...]
        D = xm.shape[1]
        g1 = gn(xm, g1w_ref[...], g1b_ref[...]).astype(jnp.bfloat16)
        g2 = gn(xa, g2w_ref[...], g2b_ref[...]).astype(jnp.bfloat16)
        qT_s[...] = jnp.concatenate(
            [jnp.swapaxes(g1, 0, 1), jnp.swapaxes(g2, 0, 1),
             jnp.zeros((D, 126), jnp.bfloat16)], axis=1)

    s128 = jnp.dot(x2_ref[0], qT_s[...], preferred_element_type=jnp.float32)
    s_ref[0] = s128[:, 0:1] + s128[:, 1:2]


def _k4_body(s_ref, idx_ref, *, L, k):
    s = s_ref[...]
    B = s.shape[0]
    col = lax.broadcasted_iota(jnp.int32, (B, L), 1)
    cols = []
    for _ in range(k):
        m = jnp.max(s, axis=1, keepdims=True)
        eq = s == m
        idx = jnp.min(jnp.where(eq, col, L), axis=1, keepdims=True)
        cols.append(idx)
        s = jnp.where(col == idx, -jnp.inf, s)
    idxmat = jnp.concatenate(cols, axis=1)
    idx_ref[...] = idxmat + lax.broadcasted_iota(jnp.int32, (B, k), 0) * L


def _k6_body(xr_ref, wlT_ref, bl_ref, wl1T_ref, bl1_ref, out_ref):
    x1 = jnp.dot(xr_ref[...].astype(jnp.bfloat16), wlT_ref[...],
                 preferred_element_type=jnp.float32) + bl_ref[...]
    out_ref[...] = jnp.dot(x1.astype(jnp.bfloat16), wl1T_ref[...],
                           preferred_element_type=jnp.float32) + bl1_ref[...]


def _sc_gather(table, idx_flat, n_rows, rows_per_worker):
    D = table.shape[1]
    n_workers = n_rows // rows_per_worker
    mesh = plsc.VectorSubcoreMesh(core_axis_name="c", subcore_axis_name="s")

    @functools.partial(
        pl.kernel, mesh=mesh,
        out_type=jax.ShapeDtypeStruct((n_rows, D), jnp.float32),
        scratch_types=[
            pltpu.VMEM((rows_per_worker,), jnp.int32),
            pltpu.VMEM((rows_per_worker, D), jnp.float32),
            pltpu.SemaphoreType.DMA,
        ],
    )
    def gather_k(table_hbm, idx_hbm, out_hbm, idx_v, rows_v, sem):
        wid = lax.axis_index("s") * 2 + lax.axis_index("c")

        @pl.when(wid < n_workers)
        def _():
            base = wid * rows_per_worker
            pltpu.sync_copy(idx_hbm.at[pl.ds(base, rows_per_worker)], idx_v)
            pltpu.async_copy(table_hbm.at[idx_v], rows_v, sem).wait()
            pltpu.sync_copy(rows_v, out_hbm.at[pl.ds(base, rows_per_worker)])

    return gather_k(table, idx_flat)


def kernel(x, W_lin, b_lin, W_lin1, b_lin1, W_max, b_max, W_aug, b_aug,
           gn1_w, gn1_b, gn2_w, gn2_b):
    B, L, D = x.shape
    NT = L // TL
    groups = 32

    wlT = W_lin.T.astype(jnp.bfloat16)
    wl1T = W_lin1.T.astype(jnp.bfloat16)
    wmT = W_max.T.astype(jnp.bfloat16)
    waT = W_aug.T.astype(jnp.bfloat16)
    row = lambda a: a.reshape(1, D)

    agg = (jnp.arange(D, dtype=jnp.int32)[:, None] // (D // groups)
           == jnp.arange(groups, dtype=jnp.int32)[None, :]).astype(jnp.float32)
    G = B * NT
    cur = lambda i: jnp.minimum(i, G - 1)
    prv = lambda i: jnp.maximum(i - NT, 0)
    s3 = pl.pallas_call(
        functools.partial(_k1s_body, G=G, NT=NT, L=L, groups=groups,
                          eps=1e-5),
        grid=(G + NT,),
        in_specs=[
            pl.BlockSpec((1, TL, D), lambda i: (cur(i) // NT, cur(i) % NT, 0)),
            pl.BlockSpec((D, D), lambda i: (0, 0)),
            pl.BlockSpec((1, D), lambda i: (0, 0)),
            pl.BlockSpec((D, D), lambda i: (0, 0)),
            pl.BlockSpec((1, D), lambda i: (0, 0)),
            pl.BlockSpec((D, D), lambda i: (0, 0)),
            pl.BlockSpec((1, D), lambda i: (0, 0)),
            pl.BlockSpec((D, D), lambda i: (0, 0)),
            pl.BlockSpec((1, D), lambda i: (0, 0)),
            pl.BlockSpec((1, D), lambda i: (0, 0)),
            pl.BlockSpec((1, D), lambda i: (0, 0)),
            pl.BlockSpec((1, D), lambda i: (0, 0)),
            pl.BlockSpec((1, D), lambda i: (0, 0)),
            pl.BlockSpec((D, groups), lambda i: (0, 0)),
            pl.BlockSpec((groups, D), lambda i: (0, 0)),
        ],
        out_specs=pl.BlockSpec((1, TL, 1),
                               lambda i: (prv(i) // NT, prv(i) % NT, 0)),
        out_shape=jax.ShapeDtypeStruct((B, L, 1), jnp.float32),
        scratch_shapes=[
            pltpu.VMEM((NT, TL, D), jnp.bfloat16),
            pltpu.VMEM((1, D), jnp.float32),
            pltpu.VMEM((1, D), jnp.float32),
            pltpu.VMEM((D, 128), jnp.bfloat16),
        ],
        compiler_params=pltpu.CompilerParams(
            dimension_semantics=("arbitrary",)),
    )(x, wlT, row(b_lin), wl1T, row(b_lin1), wmT, row(b_max), waT,
      row(b_aug), row(gn1_w), row(gn1_b), row(gn2_w), row(gn2_b), agg, agg.T)

    flat_idx = pl.pallas_call(
        functools.partial(_k4_body, L=L, k=K),
        out_shape=jax.ShapeDtypeStruct((B, K), jnp.int32),
    )(s3.reshape(B, L))

    x_rows = _sc_gather(x.reshape(B * L, D), flat_idx.reshape(B * K),
                        n_rows=B * K, rows_per_worker=8)
    selected = pl.pallas_call(
        _k6_body,
        out_shape=jax.ShapeDtypeStruct((B * K, D), jnp.float32),
    )(x_rows, wlT, row(b_lin), wl1T, row(b_lin1))
    return selected.reshape(B, K, D)


# speedup vs baseline: 1.0476x; 1.0476x over previous
import functools

import jax
import jax.numpy as jnp
from jax import lax
from jax.experimental import pallas as pl
from jax.experimental.pallas import tpu as pltpu
from jax.experimental.pallas import tpu_sc as plsc

K = 20
TL = 2048


def _k1s_body(x_ref, wlT_ref, bl_ref, wl1T_ref, bl1_ref,
              wmT_ref, bm_ref, waT_ref, ba_ref,
              g1w_ref, g1b_ref, g2w_ref, g2b_ref, agg_ref, aggT_ref,
              s_ref, x2_s, sum_s, max_s, qT_s, *, G, NT, L, groups, eps):
    i = pl.program_id(0)
    t = i % NT
    cs = 1.0 / (agg_ref.shape[0] // groups)
    hi = lax.Precision.HIGHEST

    @pl.when((t == 0) & (i >= NT))
    def _():
        def gn(y, w, b_):
            gs = jnp.dot(y, agg_ref[...], precision=hi,
                         preferred_element_type=jnp.float32) * cs
            gss = jnp.dot(y * y, agg_ref[...], precision=hi,
                          preferred_element_type=jnp.float32) * cs
            rstd = lax.rsqrt(gss - gs * gs + eps)
            mean_f = jnp.dot(gs, aggT_ref[...], precision=hi,
                             preferred_element_type=jnp.float32)
            rstd_f = jnp.dot(rstd, aggT_ref[...], precision=hi,
                             preferred_element_type=jnp.float32)
            return (y - mean_f) * rstd_f * w + b_

        mean = (sum_s[...] * (1.0 / L)).astype(jnp.bfloat16)
        xm = jnp.dot(mean, wmT_ref[...],
                     preferred_element_type=jnp.float32) + bm_ref[...]
        xa = jnp.dot(max_s[...].astype(jnp.bfloat16), waT_ref[...],
                     preferred_element_type=jnp.float32) + ba_ref[...]
        D = xm.shape[1]
        g1 = gn(xm, g1w_ref[...], g1b_ref[...]).astype(jnp.bfloat16)
        g2 = gn(xa, g2w_ref[...], g2b_ref[...]).astype(jnp.bfloat16)
        qT_s[...] = jnp.concatenate(
            [jnp.swapaxes(g1, 0, 1), jnp.swapaxes(g2, 0, 1),
             jnp.zeros((D, 126), jnp.bfloat16)], axis=1)

    @pl.when(i >= NT)
    def _():
        s128 = jnp.dot(x2_s[t], qT_s[...], preferred_element_type=jnp.float32)
        s_ref[0] = s128[:, 0:1] + s128[:, 1:2]

    @pl.when(i < G)
    def _():
        x1 = jnp.dot(x_ref[0].astype(jnp.bfloat16), wlT_ref[...],
                     preferred_element_type=jnp.float32) + bl_ref[...]
        part_sum = jnp.sum(x1, axis=0, keepdims=True)
        part_max = jnp.max(x1, axis=0, keepdims=True)

        @pl.when(t == 0)
        def _():
            sum_s[...] = part_sum
            max_s[...] = part_max

        @pl.when(t != 0)
        def _():
            sum_s[...] = sum_s[...] + part_sum
            max_s[...] = jnp.maximum(max_s[...], part_max)

        x2 = jnp.dot(x1.astype(jnp.bfloat16), wl1T_ref[...],
                     preferred_element_type=jnp.float32) + bl1_ref[...]
        x2_s[t] = x2.astype(jnp.bfloat16)


def _k4_body(s_ref, idx_ref, *, L, k):
    s = s_ref[...]
    B = s.shape[0]
    col = lax.broadcasted_iota(jnp.int32, (B, L), 1)
    cols = []
    for _ in range(k):
        m = jnp.max(s, axis=1, keepdims=True)
        eq = s == m
        idx = jnp.min(jnp.where(eq, col, L), axis=1, keepdims=True)
        cols.append(idx)
        s = jnp.where(col == idx, -jnp.inf, s)
    idxmat = jnp.concatenate(cols, axis=1)
    idx_ref[...] = idxmat + lax.broadcasted_iota(jnp.int32, (B, k), 0) * L


def _k6_body(xr_ref, wlT_ref, bl_ref, wl1T_ref, bl1_ref, out_ref):
    x1 = jnp.dot(xr_ref[...].astype(jnp.bfloat16), wlT_ref[...],
                 preferred_element_type=jnp.float32) + bl_ref[...]
    out_ref[...] = jnp.dot(x1.astype(jnp.bfloat16), wl1T_ref[...],
                           preferred_element_type=jnp.float32) + bl1_ref[...]


def _sc_gather(table, idx_flat, n_rows, rows_per_worker):
    D = table.shape[1]
    n_workers = n_rows // rows_per_worker
    mesh = plsc.VectorSubcoreMesh(core_axis_name="c", subcore_axis_name="s")

    @functools.partial(
        pl.kernel, mesh=mesh,
        out_type=jax.ShapeDtypeStruct((n_rows, D), jnp.float32),
        scratch_types=[
            pltpu.VMEM((rows_per_worker,), jnp.int32),
            pltpu.VMEM((rows_per_worker, D), jnp.float32),
            pltpu.SemaphoreType.DMA,
        ],
    )
    def gather_k(table_hbm, idx_hbm, out_hbm, idx_v, rows_v, sem):
        wid = lax.axis_index("s") * 2 + lax.axis_index("c")

        @pl.when(wid < n_workers)
        def _():
            base = wid * rows_per_worker
            pltpu.sync_copy(idx_hbm.at[pl.ds(base, rows_per_worker)], idx_v)
            pltpu.async_copy(table_hbm.at[idx_v], rows_v, sem).wait()
            pltpu.sync_copy(rows_v, out_hbm.at[pl.ds(base, rows_per_worker)])

    return gather_k(table, idx_flat)


def kernel(x, W_lin, b_lin, W_lin1, b_lin1, W_max, b_max, W_aug, b_aug,
           gn1_w, gn1_b, gn2_w, gn2_b):
    B, L, D = x.shape
    NT = L // TL
    groups = 32

    wlT = W_lin.T.astype(jnp.bfloat16)
    wl1T = W_lin1.T.astype(jnp.bfloat16)
    wmT = W_max.T.astype(jnp.bfloat16)
    waT = W_aug.T.astype(jnp.bfloat16)
    row = lambda a: a.reshape(1, D)

    agg = (jnp.arange(D, dtype=jnp.int32)[:, None] // (D // groups)
           == jnp.arange(groups, dtype=jnp.int32)[None, :]).astype(jnp.float32)
    G = B * NT
    cur = lambda i: jnp.minimum(i, G - 1)
    prv = lambda i: jnp.maximum(i - NT, 0)
    s3 = pl.pallas_call(
        functools.partial(_k1s_body, G=G, NT=NT, L=L, groups=groups,
                          eps=1e-5),
        grid=(G + NT,),
        in_specs=[
            pl.BlockSpec((1, TL, D), lambda i: (cur(i) // NT, cur(i) % NT, 0)),
            pl.BlockSpec((D, D), lambda i: (0, 0)),
            pl.BlockSpec((1, D), lambda i: (0, 0)),
            pl.BlockSpec((D, D), lambda i: (0, 0)),
            pl.BlockSpec((1, D), lambda i: (0, 0)),
            pl.BlockSpec((D, D), lambda i: (0, 0)),
            pl.BlockSpec((1, D), lambda i: (0, 0)),
            pl.BlockSpec((D, D), lambda i: (0, 0)),
            pl.BlockSpec((1, D), lambda i: (0, 0)),
            pl.BlockSpec((1, D), lambda i: (0, 0)),
            pl.BlockSpec((1, D), lambda i: (0, 0)),
            pl.BlockSpec((1, D), lambda i: (0, 0)),
            pl.BlockSpec((1, D), lambda i: (0, 0)),
            pl.BlockSpec((D, groups), lambda i: (0, 0)),
            pl.BlockSpec((groups, D), lambda i: (0, 0)),
        ],
        out_specs=pl.BlockSpec((1, TL, 1),
                               lambda i: (prv(i) // NT, prv(i) % NT, 0)),
        out_shape=jax.ShapeDtypeStruct((B, L, 1), jnp.float32),
        scratch_shapes=[
            pltpu.VMEM((NT, TL, D), jnp.bfloat16),
            pltpu.VMEM((1, D), jnp.float32),
            pltpu.VMEM((1, D), jnp.float32),
            pltpu.VMEM((D, 128), jnp.bfloat16),
        ],
        compiler_params=pltpu.CompilerParams(
            dimension_semantics=("arbitrary",)),
    )(x, wlT, row(b_lin), wl1T, row(b_lin1), wmT, row(b_max), waT,
      row(b_aug), row(gn1_w), row(gn1_b), row(gn2_w), row(gn2_b), agg, agg.T)

    flat_idx = pl.pallas_call(
        functools.partial(_k4_body, L=L, k=K),
        out_shape=jax.ShapeDtypeStruct((B, K), jnp.int32),
    )(s3.reshape(B, L))

    x_rows = _sc_gather(x.reshape(B * L, D), flat_idx.reshape(B * K),
                        n_rows=B * K, rows_per_worker=8)
    selected = pl.pallas_call(
        _k6_body,
        out_shape=jax.ShapeDtypeStruct((B * K, D), jnp.float32),
    )(x_rows, wlT, row(b_lin), wl1T, row(b_lin1))
    return selected.reshape(B, K, D)
